# baseline (speedup 1.0000x reference)
import jax
import jax.numpy as jnp
from jax import lax
from jax.experimental import pallas as pl
from jax.experimental.pallas import tpu as pltpu
from jax.experimental.pallas import tpu_sc as plsc

N = 500000
V = 100
D = 128
NC = 2
NS = 16
NW = NC * NS
C = 200
ROWS_BIG = 16000
ROWS_SMALL = 15600
CH_BIG = ROWS_BIG // C
CH_SMALL = ROWS_SMALL // C
LOOP_ITERS = CH_BIG + 1


def _embed_body(emb_hbm, z_hbm, out_hbm, table_sp, idx_v,
                rows0, rows1, rows2,
                sg0, sg1, sg2, so0, so1, so2, sem_i):
    cid = lax.axis_index("c")
    sid = lax.axis_index("s")
    wid = sid * NC + cid

    is_big = wid < 2
    start = wid * ROWS_SMALL + jnp.minimum(wid, 2) * C * 2
    n = jnp.where(is_big, CH_BIG, CH_SMALL)

    @pl.when(is_big)
    def _():
        pltpu.async_copy(z_hbm.at[pl.ds(start, ROWS_BIG)],
                         idx_v.at[pl.ds(0, ROWS_BIG)], sem_i)

    @pl.when(jnp.logical_not(is_big))
    def _():
        pltpu.async_copy(z_hbm.at[pl.ds(start, ROWS_SMALL)],
                         idx_v.at[pl.ds(0, ROWS_SMALL)], sem_i)

    @pl.when(sid == 0)
    def _():
        pltpu.sync_copy(emb_hbm, table_sp)

    plsc.subcore_barrier()

    @pl.when(is_big)
    def _():
        pltpu.make_async_copy(z_hbm.at[pl.ds(0, ROWS_BIG)],
                              idx_v.at[pl.ds(0, ROWS_BIG)], sem_i).wait()

    @pl.when(jnp.logical_not(is_big))
    def _():
        pltpu.make_async_copy(z_hbm.at[pl.ds(0, ROWS_SMALL)],
                              idx_v.at[pl.ds(0, ROWS_SMALL)], sem_i).wait()

    rows = (rows0, rows1, rows2)
    sgs = (sg0, sg1, sg2)
    sos = (so0, so1, so2)

    def step(i, b):
        @pl.when(jnp.logical_and(i >= 3, i < n))
        def _():
            pltpu.make_async_copy(
                rows[b], out_hbm.at[pl.ds(0, C)], sos[b]).wait()

        @pl.when(i < n)
        def _():
            pltpu.async_copy(
                table_sp.at[idx_v.at[pl.ds(i * C, C)]], rows[b], sgs[b])

        bp = (b + 2) % 3
        @pl.when(jnp.logical_and(i >= 1, i - 1 < n))
        def _():
            pltpu.make_async_copy(
                table_sp.at[idx_v.at[pl.ds(0, C)]], rows[bp], sgs[bp]).wait()
            pltpu.async_copy(
                rows[bp], out_hbm.at[pl.ds(start + (i - 1) * C, C)], sos[bp])

    def loop_body(g, carry):
        i = g * 3
        step(i, 0)
        step(i + 1, 1)
        step(i + 2, 2)
        return carry

    lax.fori_loop(0, (LOOP_ITERS + 2) // 3, loop_body, 0)

    pltpu.make_async_copy(rows0, out_hbm.at[pl.ds(0, C)], so0).wait()
    pltpu.make_async_copy(rows1, out_hbm.at[pl.ds(0, C)], so1).wait()
    pltpu.make_async_copy(rows2, out_hbm.at[pl.ds(0, C)], so2).wait()


_mesh = plsc.VectorSubcoreMesh(
    core_axis_name="c", subcore_axis_name="s", num_cores=NC, num_subcores=NS
)

_embed = pl.kernel(
    _embed_body,
    out_type=jax.ShapeDtypeStruct((N, D), jnp.float32),
    mesh=_mesh,
    scratch_types=[
        pltpu.VMEM_SHARED((V, D), jnp.float32),
        pltpu.VMEM((ROWS_BIG,), jnp.int32),
        pltpu.VMEM((C, D), jnp.float32),
        pltpu.VMEM((C, D), jnp.float32),
        pltpu.VMEM((C, D), jnp.float32),
        pltpu.SemaphoreType.DMA,
        pltpu.SemaphoreType.DMA,
        pltpu.SemaphoreType.DMA,
        pltpu.SemaphoreType.DMA,
        pltpu.SemaphoreType.DMA,
        pltpu.SemaphoreType.DMA,
        pltpu.SemaphoreType.DMA,
    ],
)


@jax.jit
def kernel(dR, Z, embeddings):
    del dR
    return _embed(embeddings, Z.astype(jnp.int32))
